# speedup vs baseline: 2.4148x; 2.4148x over previous
import functools
import math

import jax
import jax.numpy as jnp
from jax import lax
from jax.experimental import pallas as pl
from jax.experimental.pallas import tpu as pltpu
from jax.experimental.pallas import tpu_sc as plsc

VOCAB = 100000
D_MODEL = 128
BATCH = 4096
SEQ = 50

NUM_CORES = 2
NUM_SUBCORES = 16
NW = NUM_CORES * NUM_SUBCORES
B_TOTAL = BATCH * SEQ
B_PER_W = B_TOTAL // NW
CHUNK = 128
N_CHUNKS = B_PER_W // CHUNK
SCALE = math.sqrt(D_MODEL)


def _sc_kernel(x_hbm, table_hbm, out_hbm, idx_v, buf_v, sem):
    wid = lax.axis_index("s") * NUM_CORES + lax.axis_index("c")
    base = wid * B_PER_W
    pltpu.sync_copy(x_hbm.at[wid], idx_v)

    def chunk_body(j, carry):
        pltpu.async_copy(table_hbm.at[idx_v.at[j]], buf_v, sem).wait()

        def scale_row(r, c2):
            for c8 in range(D_MODEL // 16):
                sl = pl.ds(c8 * 16, 16)
                buf_v[r, sl] = buf_v[r, sl] * SCALE
            return c2

        lax.fori_loop(0, CHUNK, scale_row, 0, unroll=False)
        pltpu.sync_copy(buf_v, out_hbm.at[pl.ds(base + j * CHUNK, CHUNK)])
        return carry

    lax.fori_loop(0, N_CHUNKS, chunk_body, 0, unroll=False)


@functools.partial(jax.jit)
def kernel(x, table):
    xw = x.astype(jnp.int32).reshape(NW, N_CHUNKS, CHUNK)
    mesh = plsc.VectorSubcoreMesh(core_axis_name="c", subcore_axis_name="s")
    out = pl.kernel(
        _sc_kernel,
        mesh=mesh,
        out_type=jax.ShapeDtypeStruct((B_TOTAL, D_MODEL), jnp.float32),
        scratch_types=[
            pltpu.VMEM((N_CHUNKS, CHUNK), jnp.int32),
            pltpu.VMEM((CHUNK, D_MODEL), jnp.float32),
            pltpu.SemaphoreType.DMA,
        ],
    )(xw, table)
    return out.reshape(BATCH, SEQ, D_MODEL)


# speedup vs baseline: 5.1096x; 2.1159x over previous
import functools
import math

import jax
import jax.numpy as jnp
from jax import lax
from jax.experimental import pallas as pl
from jax.experimental.pallas import tpu as pltpu
from jax.experimental.pallas import tpu_sc as plsc

VOCAB = 100000
D_MODEL = 128
BATCH = 4096
SEQ = 50

NUM_CORES = 2
NUM_SUBCORES = 16
NW = NUM_CORES * NUM_SUBCORES
B_PER_W = BATCH // NW
G = 4
N_GROUPS = B_PER_W // G
SCALE = math.sqrt(D_MODEL)


def _sc_kernel(x_hbm, table_hbm, out_hbm, idx_v, buf0, buf1, gs0, gs1, ss0, ss1):
    wid = lax.axis_index("s") * NUM_CORES + lax.axis_index("c")
    b_base = wid * B_PER_W
    pltpu.sync_copy(x_hbm.at[pl.ds(b_base, B_PER_W)], idx_v)

    def fire_gathers(g, buf, gsem):
        copies = []
        for k in range(G):
            r = g * G + k
            copies.append(
                pltpu.async_copy(table_hbm.at[idx_v.at[r]], buf.at[k], gsem))
        return copies

    def scale_buf(buf):
        def row_body(r, c2):
            for k in range(G):
                for c8 in range(D_MODEL // 16):
                    sl = pl.ds(c8 * 16, 16)
                    buf[k, r, sl] = buf[k, r, sl] * SCALE
            return c2
        lax.fori_loop(0, SEQ, row_body, 0, unroll=False)

    def drain_scatter(buf, ssem):
        pltpu.make_async_copy(buf, out_hbm.at[pl.ds(b_base, G)], ssem).wait()

    def body(i, carry):
        g0 = i * 2
        g1 = i * 2 + 1

        @pl.when(i > 0)
        def _():
            drain_scatter(buf0, ss0)
        gcopies0 = fire_gathers(g0, buf0, gs0)

        @pl.when(i > 0)
        def _():
            drain_scatter(buf1, ss1)
        gcopies1 = fire_gathers(g1, buf1, gs1)

        for c in gcopies0:
            c.wait()
        scale_buf(buf0)
        pltpu.async_copy(buf0, out_hbm.at[pl.ds(b_base + g0 * G, G)], ss0)

        for c in gcopies1:
            c.wait()
        scale_buf(buf1)
        pltpu.async_copy(buf1, out_hbm.at[pl.ds(b_base + g1 * G, G)], ss1)
        return carry

    lax.fori_loop(0, N_GROUPS // 2, body, 0, unroll=False)
    drain_scatter(buf0, ss0)
    drain_scatter(buf1, ss1)


@functools.partial(jax.jit)
def kernel(x, table):
    mesh = plsc.VectorSubcoreMesh(core_axis_name="c", subcore_axis_name="s")
    return pl.kernel(
        _sc_kernel,
        mesh=mesh,
        out_type=jax.ShapeDtypeStruct((BATCH, SEQ, D_MODEL), jnp.float32),
        scratch_types=[
            pltpu.VMEM((B_PER_W, SEQ), jnp.int32),
            pltpu.VMEM((G, SEQ, D_MODEL), jnp.float32),
            pltpu.VMEM((G, SEQ, D_MODEL), jnp.float32),
            pltpu.SemaphoreType.DMA,
            pltpu.SemaphoreType.DMA,
            pltpu.SemaphoreType.DMA,
            pltpu.SemaphoreType.DMA,
        ],
    )(x.astype(jnp.int32), table)
